# Initial kernel scaffold; baseline (speedup 1.0000x reference)
import jax
import jax.numpy as jnp
from jax.experimental import pallas as pl


def kernel(indices, items_emb):
    raise NotImplementedError("write your pallas kernel here")


# speedup vs baseline: 1.2850x; 1.2850x over previous
import functools

import jax
import jax.numpy as jnp
from jax import lax
from jax.experimental import pallas as pl
from jax.experimental.pallas import tpu as pltpu
from jax.experimental.pallas import tpu_sc as plsc

M_ITEM = 1000000
DIM = 32
B, S = 16384, 50
LANES = 128
N_ROWS = (B * S) // LANES
NW = 32
ROWS_PER_W = N_ROWS // NW
CH = 8


def _make_gather():
    mesh = plsc.VectorSubcoreMesh(core_axis_name="c", subcore_axis_name="s")

    @functools.partial(
        pl.kernel,
        mesh=mesh,
        out_type=jax.ShapeDtypeStruct((N_ROWS, LANES, DIM), jnp.float32),
        scratch_types=[
            pltpu.VMEM((CH, LANES), jnp.int32),
            pltpu.VMEM((CH, LANES, DIM), jnp.float32),
            pltpu.SemaphoreType.DMA,
        ],
        compiler_params=pltpu.CompilerParams(use_tc_tiling_on_sc=False),
    )
    def gather_kernel(table_hbm, idx_hbm, out_hbm, idx_v, rows_v, sem):
        wid = lax.axis_index("s") * 2 + lax.axis_index("c")
        base = wid * ROWS_PER_W

        def body(i, carry):
            row = base + i * CH
            pltpu.sync_copy(idx_hbm.at[pl.ds(row, CH)], idx_v)
            copies = [
                pltpu.async_copy(table_hbm.at[idx_v.at[j]], rows_v.at[j], sem)
                for j in range(CH)
            ]
            for c in copies:
                c.wait()
            pltpu.sync_copy(rows_v, out_hbm.at[pl.ds(row, CH)])
            return carry

        lax.fori_loop(0, ROWS_PER_W // CH, body, 0)

    return gather_kernel


_gather = _make_gather()


def kernel(indices, items_emb):
    idx_flat = indices.reshape(N_ROWS, LANES).astype(jnp.int32)
    out = _gather(items_emb, idx_flat)
    return out.reshape(B, S, DIM)


# speedup vs baseline: 1.3022x; 1.0134x over previous
import functools

import jax
import jax.numpy as jnp
from jax import lax
from jax.experimental import pallas as pl
from jax.experimental.pallas import tpu as pltpu
from jax.experimental.pallas import tpu_sc as plsc

M_ITEM = 1000000
DIM = 32
B, S = 16384, 50
LANES = 128
N_ROWS = (B * S) // LANES
NW = 32
ROWS_PER_W = N_ROWS // NW
CH = 10
NCH = ROWS_PER_W // CH


def _make_gather():
    mesh = plsc.VectorSubcoreMesh(core_axis_name="c", subcore_axis_name="s")

    @functools.partial(
        pl.kernel,
        mesh=mesh,
        out_type=jax.ShapeDtypeStruct((N_ROWS, LANES, DIM), jnp.float32),
        scratch_types=[
            pltpu.VMEM((2, CH, LANES), jnp.int32),
            pltpu.VMEM((2, CH, LANES, DIM), jnp.float32),
            pltpu.SemaphoreType.DMA,
            pltpu.SemaphoreType.DMA,
            pltpu.SemaphoreType.DMA,
            pltpu.SemaphoreType.DMA,
        ],
        compiler_params=pltpu.CompilerParams(use_tc_tiling_on_sc=False),
    )
    def gather_kernel(table_hbm, idx_hbm, out_hbm, idx_v, rows_v, g0, g1, o0, o1):
        gsem = (g0, g1)
        osem = (o0, o1)
        wid = lax.axis_index("s") * 2 + lax.axis_index("c")
        base = wid * ROWS_PER_W

        def fire(b, chunk):
            row = base + chunk * CH
            pltpu.sync_copy(idx_hbm.at[pl.ds(row, CH)], idx_v.at[b])
            for j in range(CH):
                pltpu.async_copy(
                    table_hbm.at[idx_v.at[b].at[j]], rows_v.at[b].at[j], gsem[b]
                )

        def drain_fire_out(b, chunk):
            row = base + chunk * CH
            pltpu.make_async_copy(
                out_hbm.at[pl.ds(0, CH)], rows_v.at[b], gsem[b]
            ).wait()
            pltpu.async_copy(rows_v.at[b], out_hbm.at[pl.ds(row, CH)], osem[b])

        def wait_out(b):
            pltpu.make_async_copy(
                rows_v.at[b], out_hbm.at[pl.ds(0, CH)], osem[b]
            ).wait()

        fire(0, 0)
        fire(1, 1)
        drain_fire_out(0, 0)

        def body(i, carry):
            c = 2 + 2 * i
            wait_out(0)
            fire(0, c)
            drain_fire_out(1, c - 1)
            wait_out(1)
            fire(1, c + 1)
            drain_fire_out(0, c)
            return carry

        lax.fori_loop(0, (NCH - 2) // 2, body, 0)

        drain_fire_out(1, NCH - 1)
        wait_out(0)
        wait_out(1)

    return gather_kernel


_gather = _make_gather()


def kernel(indices, items_emb):
    idx_flat = indices.reshape(N_ROWS, LANES).astype(jnp.int32)
    out = _gather(items_emb, idx_flat)
    return out.reshape(B, S, DIM)
